# baseline (speedup 1.0000x reference)
import functools

import jax
import jax.numpy as jnp
from jax import lax
from jax.experimental import pallas as pl
from jax.experimental.pallas import tpu as pltpu
from jax.experimental.pallas import tpu_sc as plsc

N = 10000
E = 320000
H = 128
C = 32

NC = 2
NS = 16
NW = NC * NS
KC = 128
EPTS = E // NW
NCHS = 80
PADS = NCHS * KC - EPTS
EPTD = E // NS
NCHD = -(-EPTD // KC)
PADD = NCHD * KC - EPTD
TRASH = 16
W1 = 624
ZC = 104


def _zero_buf(buf, nrows, width):
    zv = jnp.zeros((16,), jnp.float32)

    def zrow(i, _):
        for j in range(width // 16):
            buf[i, pl.ds(j * 16, 16)] = zv
        return 0

    lax.fori_loop(0, nrows, zrow, 0)


def _zero_acc_slice(zbuf, acc, s):
    base = s * W1
    for t in range(W1 // ZC):
        pltpu.sync_copy(zbuf.at[pl.ds(0, ZC)],
                        acc.at[pl.ds(base + t * ZC, ZC)])

    @pl.when(s == NS - 1)
    def _():
        pltpu.sync_copy(zbuf.at[pl.ds(0, 16)], acc.at[pl.ds(NS * W1, 16)])


def _writeout_slice(acc, out_hbm, s, pre):
    base = s * W1
    pltpu.sync_copy(acc.at[pl.ds(base, W1)],
                    out_hbm.at[pre + (pl.ds(base, W1),)])

    @pl.when(s == NS - 1)
    def _():
        pltpu.sync_copy(acc.at[pl.ds(NS * W1, 16)],
                        out_hbm.at[pre + (pl.ds(NS * W1, 16),)])


def _make_sc_segsum(Hw):
    mesh = plsc.VectorSubcoreMesh(core_axis_name="c", subcore_axis_name="s")
    HR = NCHS // 2
    HP = HR // 2

    @functools.partial(
        pl.kernel, mesh=mesh,
        out_type=jax.ShapeDtypeStruct((NC, N, Hw), jnp.float32),
        scratch_types=[
            pltpu.VMEM((HR, KC), jnp.int32),
            pltpu.VMEM((HR, KC), jnp.int32),
            pltpu.VMEM((KC, Hw), jnp.float32),
            pltpu.VMEM((KC, Hw), jnp.float32),
            pltpu.VMEM_SHARED((N + TRASH, Hw), jnp.float32),
            pltpu.SemaphoreType.DMA,
            pltpu.SemaphoreType.DMA,
            pltpu.SemaphoreType.DMA,
            pltpu.SemaphoreType.DMA,
        ],
    )
    def k(y_hbm, src_hbm, dst_hbm, out_hbm, srcv, dstv, rows0, rows1,
          acc, g0, g1, s0, s1):
        c = lax.axis_index("c")
        s = lax.axis_index("s")
        w = c * NS + s

        _zero_buf(rows0, KC, Hw)
        _zero_acc_slice(rows0, acc, s)
        plsc.subcore_barrier()

        for half in range(2):
            pltpu.sync_copy(src_hbm.at[w, pl.ds(half * HR, HR)], srcv)
            pltpu.sync_copy(dst_hbm.at[w, pl.ds(half * HR, HR)], dstv)

            def pair(i, _):
                j0 = 2 * i
                j1 = 2 * i + 1
                h0 = pltpu.async_copy(y_hbm.at[srcv.at[j0]], rows0, g0)
                h1 = pltpu.async_copy(y_hbm.at[srcv.at[j1]], rows1, g1)
                h0.wait()
                c0 = pltpu.async_copy(rows0, acc.at[dstv.at[j0]], s0,
                                      add=True)
                h1.wait()
                c1 = pltpu.async_copy(rows1, acc.at[dstv.at[j1]], s1,
                                      add=True)
                c0.wait()
                c1.wait()
                return 0

            lax.fori_loop(0, HP, pair, 0)
        plsc.subcore_barrier()
        _writeout_slice(acc, out_hbm, s, (c,))

    return k


def _make_sc_deg():
    mesh = plsc.VectorSubcoreMesh(core_axis_name="c", subcore_axis_name="s")

    @functools.partial(
        pl.kernel, mesh=mesh,
        out_type=jax.ShapeDtypeStruct((N, H), jnp.float32),
        scratch_types=[
            pltpu.VMEM((NCHD, KC), jnp.int32),
            pltpu.VMEM((KC, H), jnp.float32),
            pltpu.VMEM_SHARED((N + TRASH, H), jnp.float32),
        ],
    )
    def k(dst_hbm, out_hbm, dstv, ones, acc):
        c = lax.axis_index("c")
        s = lax.axis_index("s")

        @pl.when(c == 0)
        def _():
            _zero_buf(ones, KC, H)
            _zero_acc_slice(ones, acc, s)
            ov = jnp.ones((16,), jnp.float32)

            def orow(i, _):
                for j in range(H // 16):
                    ones[i, pl.ds(j * 16, 16)] = ov
                return 0

            lax.fori_loop(0, KC, orow, 0)
            pltpu.sync_copy(dst_hbm.at[s], dstv)
            plsc.subcore_barrier()

            def step(j, _):
                pltpu.sync_copy(ones, acc.at[dstv.at[j]], add=True)
                return 0

            lax.fori_loop(0, NCHD, step, 0)
            plsc.subcore_barrier()
            _writeout_slice(acc, out_hbm, s, ())

    return k


BN = 1000


def _proj_body_plain(a_ref, wr_ref, wo_ref, y_ref, r_ref):
    a = a_ref[...]
    y_ref[...] = jnp.dot(a, wr_ref[...], preferred_element_type=jnp.float32)
    r_ref[...] = jnp.dot(a, wo_ref[...], preferred_element_type=jnp.float32)


def _proj_body_affine(a_ref, stats_ref, g_ref, b_ref, wr_ref, wo_ref,
                      y_ref, r_ref):
    st = stats_ref[...]
    m = st[0:1] / N
    var = st[1:2] / N - m * m
    scale = g_ref[...] * lax.rsqrt(var + 1e-5)
    shift = b_ref[...] - m * scale
    a = a_ref[...] * scale + shift
    y_ref[...] = jnp.dot(a, wr_ref[...], preferred_element_type=jnp.float32)
    r_ref[...] = jnp.dot(a, wo_ref[...], preferred_element_type=jnp.float32)


def _make_project(Din, Dy, Dr, affine):
    full = lambda shape: pl.BlockSpec(shape, lambda i: (0,) * len(shape))
    in_specs = [pl.BlockSpec((BN, Din), lambda i: (i, 0))]
    if affine:
        in_specs += [full((2, Din)), full((1, Din)), full((1, Din))]
    in_specs += [full((Din, Dy)), full((Din, Dr))]
    return pl.pallas_call(
        _proj_body_affine if affine else _proj_body_plain,
        grid=(N // BN,),
        in_specs=in_specs,
        out_specs=[pl.BlockSpec((BN, Dy), lambda i: (i, 0)),
                   pl.BlockSpec((BN, Dr), lambda i: (i, 0))],
        out_shape=[jax.ShapeDtypeStruct((N, Dy), jnp.float32),
                   jax.ShapeDtypeStruct((N, Dr), jnp.float32)],
    )


def _combine_body(s_ref, deg_ref, r_ref, a_ref, stats_ref):
    ssum = s_ref[0] + s_ref[1]
    d = deg_ref[:, 0:1]
    inv = 1.0 / jnp.maximum(d, 1.0)
    a = jnp.maximum(ssum * inv + r_ref[...], 0.0)
    a_ref[...] = a
    upd = jnp.concatenate(
        [jnp.sum(a, axis=0, keepdims=True),
         jnp.sum(a * a, axis=0, keepdims=True)], axis=0)
    i = pl.program_id(0)

    @pl.when(i == 0)
    def _():
        stats_ref[...] = upd

    @pl.when(i > 0)
    def _():
        stats_ref[...] += upd


def _make_combine(Hw):
    return pl.pallas_call(
        _combine_body,
        grid=(N // BN,),
        in_specs=[
            pl.BlockSpec((NC, BN, Hw), lambda i: (0, i, 0)),
            pl.BlockSpec((BN, H), lambda i: (i, 0)),
            pl.BlockSpec((BN, Hw), lambda i: (i, 0)),
        ],
        out_specs=[
            pl.BlockSpec((BN, Hw), lambda i: (i, 0)),
            pl.BlockSpec((2, Hw), lambda i: (0, 0)),
        ],
        out_shape=[
            jax.ShapeDtypeStruct((N, Hw), jnp.float32),
            jax.ShapeDtypeStruct((2, Hw), jnp.float32),
        ],
    )


def _final_body(s_ref, deg_ref, r_ref, o_ref):
    u = (s_ref[0] + s_ref[1])[:, :C]
    d = deg_ref[:, 0:1]
    inv = 1.0 / jnp.maximum(d, 1.0)
    u = u * inv + r_ref[...]
    mx = jnp.max(u, axis=1, keepdims=True)
    lse = jnp.log(jnp.sum(jnp.exp(u - mx), axis=1, keepdims=True)) + mx
    o_ref[...] = u - lse


def _make_final():
    return pl.pallas_call(
        _final_body,
        grid=(N // BN,),
        in_specs=[
            pl.BlockSpec((NC, BN, H), lambda i: (0, i, 0)),
            pl.BlockSpec((BN, H), lambda i: (i, 0)),
            pl.BlockSpec((BN, C), lambda i: (i, 0)),
        ],
        out_specs=pl.BlockSpec((BN, C), lambda i: (i, 0)),
        out_shape=jax.ShapeDtypeStruct((N, C), jnp.float32),
    )


def _pad_edges(idx, groups, pad, width=KC):
    g = idx.reshape(groups, E // groups)
    return jnp.concatenate([g, pad], axis=1).reshape(groups, -1, width)


def _trash_pad(groups, pad_slots):
    return jnp.broadcast_to(
        N + (jnp.arange(pad_slots, dtype=jnp.int32) % 8), (groups, pad_slots))


def kernel(x, edge_index, W_rel_p, W_root_p, g0, b0,
           W_rel_1, W_root_1, g1, b1,
           W_rel_2, W_root_2, g2, b2,
           W_rel_3, W_root_3, g3, b3,
           W_rel_f, W_root_f):
    src32 = _pad_edges(edge_index[0], NW, jnp.zeros((NW, PADS), jnp.int32))
    dst32 = _pad_edges(edge_index[1], NW, _trash_pad(NW, PADS))
    dst16 = _pad_edges(edge_index[1], NS, _trash_pad(NS, PADD))

    sc_h = _make_sc_segsum(H)
    sc_deg = _make_sc_deg()
    proj0 = _make_project(H, H, H, False)
    proj_h = _make_project(H, H, H, True)
    proj_f = _make_project(H, H, C, True)
    wrf_pad = jnp.concatenate(
        [W_rel_f, jnp.zeros((H, H - C), jnp.float32)], axis=1)
    combine = _make_combine(H)
    final = _make_final()

    deg = sc_deg(dst16)
    y, r = proj0(x, W_rel_p, W_root_p)
    s = sc_h(y, src32, dst32)
    a, stats = combine(s, deg, r)
    for (g, b, Wr, Wo) in ((g0, b0, W_rel_1, W_root_1),
                           (g1, b1, W_rel_2, W_root_2),
                           (g2, b2, W_rel_3, W_root_3)):
        y, r = proj_h(a, stats, g.reshape(1, H), b.reshape(1, H), Wr, Wo)
        s = sc_h(y, src32, dst32)
        a, stats = combine(s, deg, r)
    y, r = proj_f(a, stats, g3.reshape(1, H), b3.reshape(1, H),
                  wrf_pad, W_root_f)
    s = sc_h(y, src32, dst32)
    return final(s, deg, r)


# speedup vs baseline: 1.4075x; 1.4075x over previous
import functools

import jax
import jax.numpy as jnp
from jax import lax
from jax.experimental import pallas as pl
from jax.experimental.pallas import tpu as pltpu
from jax.experimental.pallas import tpu_sc as plsc

N = 10000
E = 320000
H = 128
C = 32

NC = 2
NS = 16
NW = NC * NS
KC = 128
EPTS = E // NW
NCHS = -(-EPTS // KC)
PADS = NCHS * KC - EPTS
EPTD = E // NS
NCHD = -(-EPTD // KC)
PADD = NCHD * KC - EPTD
TRASH = 16
W1 = 624
ZC = 104


def _zero_buf(buf, nrows, width):
    zv = jnp.zeros((16,), jnp.float32)

    def zrow(i, _):
        for j in range(width // 16):
            buf[i, pl.ds(j * 16, 16)] = zv
        return 0

    lax.fori_loop(0, nrows, zrow, 0)


def _zero_acc_slice(zbuf, acc, s):
    base = s * W1
    for t in range(W1 // ZC):
        pltpu.sync_copy(zbuf.at[pl.ds(0, ZC)],
                        acc.at[pl.ds(base + t * ZC, ZC)])

    @pl.when(s == NS - 1)
    def _():
        pltpu.sync_copy(zbuf.at[pl.ds(0, 16)], acc.at[pl.ds(NS * W1, 16)])


def _writeout_slice(acc, out_hbm, s, pre):
    base = s * W1
    pltpu.sync_copy(acc.at[pl.ds(base, W1)],
                    out_hbm.at[pre + (pl.ds(base, W1),)])

    @pl.when(s == NS - 1)
    def _():
        pltpu.sync_copy(acc.at[pl.ds(NS * W1, 16)],
                        out_hbm.at[pre + (pl.ds(NS * W1, 16),)])


def _make_sc_segsum(Hw):
    mesh = plsc.VectorSubcoreMesh(core_axis_name="c", subcore_axis_name="s")

    @functools.partial(
        pl.kernel, mesh=mesh,
        out_type=jax.ShapeDtypeStruct((NC, N, Hw), jnp.float32),
        scratch_types=[
            pltpu.VMEM((NCHS, KC), jnp.int32),
            pltpu.VMEM((NCHS, KC), jnp.int32),
            pltpu.VMEM((KC, Hw), jnp.float32),
            pltpu.VMEM_SHARED((N + TRASH, Hw), jnp.float32),
            pltpu.SemaphoreType.DMA,
        ],
    )
    def k(y_hbm, src_hbm, dst_hbm, out_hbm, srcv, dstv, rows, acc, sem):
        c = lax.axis_index("c")
        s = lax.axis_index("s")
        w = c * NS + s

        _zero_buf(rows, KC, Hw)
        _zero_acc_slice(rows, acc, s)
        pltpu.sync_copy(src_hbm.at[w], srcv)
        pltpu.sync_copy(dst_hbm.at[w], dstv)
        plsc.subcore_barrier()

        def step(j, _):
            pltpu.async_copy(y_hbm.at[srcv.at[j]], rows, sem).wait()
            pltpu.sync_copy(rows, acc.at[dstv.at[j]], add=True)
            return 0

        lax.fori_loop(0, NCHS, step, 0)
        plsc.subcore_barrier()
        _writeout_slice(acc, out_hbm, s, (c,))

    return k


def _make_sc_deg():
    mesh = plsc.VectorSubcoreMesh(core_axis_name="c", subcore_axis_name="s")

    @functools.partial(
        pl.kernel, mesh=mesh,
        out_type=jax.ShapeDtypeStruct((NC, N, H), jnp.float32),
        scratch_types=[
            pltpu.VMEM((NCHS, KC), jnp.int32),
            pltpu.VMEM((KC, H), jnp.float32),
            pltpu.VMEM_SHARED((N + TRASH, H), jnp.float32),
        ],
    )
    def k(dst_hbm, out_hbm, dstv, ones, acc):
        c = lax.axis_index("c")
        s = lax.axis_index("s")
        w = c * NS + s

        _zero_buf(ones, KC, H)
        _zero_acc_slice(ones, acc, s)
        ov = jnp.ones((16,), jnp.float32)

        def orow(i, _):
            for j in range(H // 16):
                ones[i, pl.ds(j * 16, 16)] = ov
            return 0

        lax.fori_loop(0, KC, orow, 0)
        pltpu.sync_copy(dst_hbm.at[w], dstv)
        plsc.subcore_barrier()

        def step(j, _):
            pltpu.sync_copy(ones, acc.at[dstv.at[j]], add=True)
            return 0

        lax.fori_loop(0, NCHS, step, 0)
        plsc.subcore_barrier()
        _writeout_slice(acc, out_hbm, s, (c,))

    return k


BN = 1000


def _proj_body_plain(a_ref, wr_ref, wo_ref, y_ref, r_ref):
    a = a_ref[...]
    y_ref[...] = jnp.dot(a, wr_ref[...], preferred_element_type=jnp.float32)
    r_ref[...] = jnp.dot(a, wo_ref[...], preferred_element_type=jnp.float32)


def _proj_body_affine(a_ref, stats_ref, g_ref, b_ref, wr_ref, wo_ref,
                      y_ref, r_ref):
    st = stats_ref[...]
    m = st[0:1] / N
    var = st[1:2] / N - m * m
    scale = g_ref[...] * lax.rsqrt(var + 1e-5)
    shift = b_ref[...] - m * scale
    a = a_ref[...] * scale + shift
    y_ref[...] = jnp.dot(a, wr_ref[...], preferred_element_type=jnp.float32)
    r_ref[...] = jnp.dot(a, wo_ref[...], preferred_element_type=jnp.float32)


def _make_project(Din, Dy, Dr, affine):
    full = lambda shape: pl.BlockSpec(shape, lambda i: (0,) * len(shape))
    in_specs = [pl.BlockSpec((BN, Din), lambda i: (i, 0))]
    if affine:
        in_specs += [full((2, Din)), full((1, Din)), full((1, Din))]
    in_specs += [full((Din, Dy)), full((Din, Dr))]
    return pl.pallas_call(
        _proj_body_affine if affine else _proj_body_plain,
        grid=(N // BN,),
        in_specs=in_specs,
        out_specs=[pl.BlockSpec((BN, Dy), lambda i: (i, 0)),
                   pl.BlockSpec((BN, Dr), lambda i: (i, 0))],
        out_shape=[jax.ShapeDtypeStruct((N, Dy), jnp.float32),
                   jax.ShapeDtypeStruct((N, Dr), jnp.float32)],
    )


def _combine_body(s_ref, deg_ref, r_ref, a_ref, stats_ref):
    ssum = s_ref[0] + s_ref[1]
    d = deg_ref[0, :, 0:1] + deg_ref[1, :, 0:1]
    inv = 1.0 / jnp.maximum(d, 1.0)
    a = jnp.maximum(ssum * inv + r_ref[...], 0.0)
    a_ref[...] = a
    upd = jnp.concatenate(
        [jnp.sum(a, axis=0, keepdims=True),
         jnp.sum(a * a, axis=0, keepdims=True)], axis=0)
    i = pl.program_id(0)

    @pl.when(i == 0)
    def _():
        stats_ref[...] = upd

    @pl.when(i > 0)
    def _():
        stats_ref[...] += upd


def _make_combine(Hw):
    return pl.pallas_call(
        _combine_body,
        grid=(N // BN,),
        in_specs=[
            pl.BlockSpec((NC, BN, Hw), lambda i: (0, i, 0)),
            pl.BlockSpec((NC, BN, H), lambda i: (0, i, 0)),
            pl.BlockSpec((BN, Hw), lambda i: (i, 0)),
        ],
        out_specs=[
            pl.BlockSpec((BN, Hw), lambda i: (i, 0)),
            pl.BlockSpec((2, Hw), lambda i: (0, 0)),
        ],
        out_shape=[
            jax.ShapeDtypeStruct((N, Hw), jnp.float32),
            jax.ShapeDtypeStruct((2, Hw), jnp.float32),
        ],
    )


def _final_body(s_ref, deg_ref, r_ref, o_ref):
    u = (s_ref[0] + s_ref[1])[:, :C]
    d = deg_ref[0, :, 0:1] + deg_ref[1, :, 0:1]
    inv = 1.0 / jnp.maximum(d, 1.0)
    u = u * inv + r_ref[...]
    mx = jnp.max(u, axis=1, keepdims=True)
    lse = jnp.log(jnp.sum(jnp.exp(u - mx), axis=1, keepdims=True)) + mx
    o_ref[...] = u - lse


def _make_final():
    return pl.pallas_call(
        _final_body,
        grid=(N // BN,),
        in_specs=[
            pl.BlockSpec((NC, BN, H), lambda i: (0, i, 0)),
            pl.BlockSpec((NC, BN, H), lambda i: (0, i, 0)),
            pl.BlockSpec((BN, C), lambda i: (i, 0)),
        ],
        out_specs=pl.BlockSpec((BN, C), lambda i: (i, 0)),
        out_shape=jax.ShapeDtypeStruct((N, C), jnp.float32),
    )


def _pad_edges(idx, groups, pad):
    g = idx.reshape(groups, E // groups)
    return jnp.concatenate([g, pad], axis=1).reshape(groups, -1, KC)


def _trash_pad(groups, pad_slots):
    return jnp.broadcast_to(
        N + (jnp.arange(pad_slots, dtype=jnp.int32) % 8), (groups, pad_slots))


def kernel(x, edge_index, W_rel_p, W_root_p, g0, b0,
           W_rel_1, W_root_1, g1, b1,
           W_rel_2, W_root_2, g2, b2,
           W_rel_3, W_root_3, g3, b3,
           W_rel_f, W_root_f):
    src32 = _pad_edges(edge_index[0], NW, jnp.zeros((NW, PADS), jnp.int32))
    dst32 = _pad_edges(edge_index[1], NW, _trash_pad(NW, PADS))

    sc_h = _make_sc_segsum(H)
    sc_deg = _make_sc_deg()
    proj0 = _make_project(H, H, H, False)
    proj_h = _make_project(H, H, H, True)
    proj_f = _make_project(H, H, C, True)
    wrf_pad = jnp.concatenate(
        [W_rel_f, jnp.zeros((H, H - C), jnp.float32)], axis=1)
    combine = _make_combine(H)
    final = _make_final()

    deg = sc_deg(dst32)
    y, r = proj0(x, W_rel_p, W_root_p)
    s = sc_h(y, src32, dst32)
    a, stats = combine(s, deg, r)
    for (g, b, Wr, Wo) in ((g0, b0, W_rel_1, W_root_1),
                           (g1, b1, W_rel_2, W_root_2),
                           (g2, b2, W_rel_3, W_root_3)):
        y, r = proj_h(a, stats, g.reshape(1, H), b.reshape(1, H), Wr, Wo)
        s = sc_h(y, src32, dst32)
        a, stats = combine(s, deg, r)
    y, r = proj_f(a, stats, g3.reshape(1, H), b3.reshape(1, H),
                  wrf_pad, W_root_f)
    s = sc_h(y, src32, dst32)
    return final(s, deg, r)
